# baseline (speedup 1.0000x reference)
import functools
import math

import jax
import jax.numpy as jnp
from jax import lax
from jax.experimental import pallas as pl
from jax.experimental.pallas import tpu as pltpu
from jax.experimental.pallas import tpu_sc as plsc

NUM_HEADS = 16
NUM_BUCKETS = 32
MAX_DISTANCE = 128

SEQ = 2048
KEY = 2048
TBL = 4088
TBLP = 4352

QT = SEQ // 8
KT = KEY // 128

NC = 2
NS = 16
NW = NC * NS
QT_PER_W = NUM_HEADS * QT // NW
GRP = 4
GRP_STEPS = QT_PER_W // GRP


def _table_body(delta_ref, wt_ref, out_ref):
    delta = delta_ref[0]
    col = lax.broadcasted_iota(jnp.int32, (8, TBLP), 1)
    row = lax.broadcasted_iota(jnp.int32, (8, TBLP), 0)
    d = col + (row - 24) - (KEY - 1) + delta
    half = NUM_BUCKETS // 2
    rel = jnp.where(d > 0, half, 0)
    ad = jnp.abs(d)
    max_exact = half // 2
    rp_safe = jnp.maximum(ad, 1)
    lg = max_exact + (
        jnp.log(rp_safe.astype(jnp.float32) / max_exact)
        / math.log(MAX_DISTANCE / max_exact)
        * (half - max_exact)
    ).astype(jnp.int32)
    lg = jnp.minimum(lg, half - 1)
    bkt = rel + jnp.where(ad < max_exact, ad, lg)
    bi = lax.broadcasted_iota(jnp.int32, (NUM_BUCKETS, TBLP), 0)
    for s in range(8):
        onehot = (bkt[s : s + 1, :] == bi).astype(jnp.float32)
        vals = lax.dot_general(
            wt_ref[...], onehot,
            (((1,), (0,)), ((), ())),
            preferred_element_type=jnp.float32,
            precision=lax.Precision.HIGHEST,
        )
        out_ref[s] = vals


def _build_table(weight, delta):
    wt = weight.T
    delta_arr = jnp.asarray(delta, jnp.int32).reshape(1)
    return pl.pallas_call(
        _table_body,
        out_shape=jax.ShapeDtypeStruct((8, NUM_HEADS, TBLP), jnp.float32),
        in_specs=[
            pl.BlockSpec(memory_space=pltpu.SMEM),
            pl.BlockSpec(memory_space=pltpu.VMEM),
        ],
        out_specs=pl.BlockSpec(memory_space=pltpu.VMEM),
    )(delta_arr, wt)


def _expand_body(tbl_hbm, out_hbm, tbl_v, sem):
    wid = lax.axis_index("s") * NC + lax.axis_index("c")
    h = wid // 2
    qt0 = (wid % 2) * QT_PER_W
    stage = []
    for g in range(GRP):
        for u in range(8):
            stage.append(
                pltpu.async_copy(
                    tbl_hbm.at[7 - u, h, pl.ds(24 - 8 * g, TBL)],
                    tbl_v.at[g, 0, u],
                    sem,
                )
            )
    for c in stage:
        c.wait()

    def step(t, carry):
        qtg = qt0 + GRP * t
        base = (KEY - 8) - 8 * qtg
        copies = []
        for kt in range(KT):
            copies.append(
                pltpu.async_copy(
                    tbl_v.at[:, :, :, pl.ds(base + 128 * kt, 128)],
                    out_hbm.at[h, pl.ds(qtg, GRP), pl.ds(kt, 1)],
                    sem,
                )
            )
        for c in copies:
            c.wait()
        return carry

    lax.fori_loop(0, GRP_STEPS, step, 0)


def _expand(tbl):
    mesh = plsc.VectorSubcoreMesh(core_axis_name="c", subcore_axis_name="s")
    run = functools.partial(
        pl.kernel,
        mesh=mesh,
        out_type=jax.ShapeDtypeStruct((NUM_HEADS, QT, KT, 8, 128), jnp.float32),
        scratch_types=[
            pltpu.VMEM((GRP, 1, 8, TBL), jnp.float32),
            pltpu.SemaphoreType.DMA,
        ],
        compiler_params=pltpu.CompilerParams(use_tc_tiling_on_sc=False),
    )(_expand_body)
    return run(tbl)


def kernel(weight, seq_length, key_length):
    delta = key_length - seq_length
    tbl = _build_table(weight, delta)
    out5 = _expand(tbl)
    return out5.transpose(0, 1, 3, 2, 4).reshape(NUM_HEADS, SEQ, KEY)


# speedup vs baseline: 1.0283x; 1.0283x over previous
import functools
import math

import jax
import jax.numpy as jnp
from jax import lax
from jax.experimental import pallas as pl
from jax.experimental.pallas import tpu as pltpu
from jax.experimental.pallas import tpu_sc as plsc

NUM_HEADS = 16
NUM_BUCKETS = 32
MAX_DISTANCE = 128

SEQ = 2048
KEY = 2048
TBL = 4224
TBLP = 4352

QT = SEQ // 8
KT = KEY // 128

NC = 2
NS = 16
NW = NC * NS
QT_PER_W = NUM_HEADS * QT // NW
PAIR_STEPS = QT_PER_W // 2


def _table_body(delta_ref, wt_ref, out_ref):
    delta = delta_ref[0]
    col = lax.broadcasted_iota(jnp.int32, (8, TBLP), 1)
    row = lax.broadcasted_iota(jnp.int32, (8, TBLP), 0)
    d = col + (row - 8) - (KEY - 1) + delta
    half = NUM_BUCKETS // 2
    rel = jnp.where(d > 0, half, 0)
    ad = jnp.abs(d)
    max_exact = half // 2
    rp_safe = jnp.maximum(ad, 1)
    lg = max_exact + (
        jnp.log(rp_safe.astype(jnp.float32) / max_exact)
        / math.log(MAX_DISTANCE / max_exact)
        * (half - max_exact)
    ).astype(jnp.int32)
    lg = jnp.minimum(lg, half - 1)
    bkt = rel + jnp.where(ad < max_exact, ad, lg)
    bi = lax.broadcasted_iota(jnp.int32, (NUM_BUCKETS, TBLP), 0)
    for s in range(8):
        onehot = (bkt[s : s + 1, :] == bi).astype(jnp.float32)
        vals = lax.dot_general(
            wt_ref[...], onehot,
            (((1,), (0,)), ((), ())),
            preferred_element_type=jnp.float32,
            precision=lax.Precision.HIGHEST,
        )
        out_ref[s] = vals


def _build_table(weight, delta):
    wt = weight.T
    delta_arr = jnp.asarray(delta, jnp.int32).reshape(1)
    return pl.pallas_call(
        _table_body,
        out_shape=jax.ShapeDtypeStruct((8, NUM_HEADS, TBLP), jnp.float32),
        in_specs=[
            pl.BlockSpec(memory_space=pltpu.SMEM),
            pl.BlockSpec(memory_space=pltpu.VMEM),
        ],
        out_specs=pl.BlockSpec(memory_space=pltpu.VMEM),
    )(delta_arr, wt)


def _expand_body(tbl_hbm, out_hbm, tbl_v, sem):
    wid = lax.axis_index("s") * NC + lax.axis_index("c")
    h = wid // 2
    qt0 = (wid % 2) * QT_PER_W
    stage = []
    for g in range(2):
        for u in range(8):
            stage.append(
                pltpu.async_copy(
                    tbl_hbm.at[7 - u, h, pl.ds(8 - 8 * g, TBL)],
                    tbl_v.at[g, 0, u],
                    sem,
                )
            )
    for c in stage:
        c.wait()

    def step(t, carry):
        qtg = qt0 + 2 * t
        base = (KEY - 8) - 8 * qtg
        copies = []
        for kt in range(KT):
            copies.append(
                pltpu.async_copy(
                    tbl_v.at[:, :, :, pl.ds(base + 128 * kt, 128)],
                    out_hbm.at[h, pl.ds(qtg, 2), pl.ds(kt, 1)],
                    sem,
                )
            )
        for c in copies:
            c.wait()
        return carry

    lax.fori_loop(0, PAIR_STEPS, step, 0)


def _expand(tbl):
    mesh = plsc.VectorSubcoreMesh(core_axis_name="c", subcore_axis_name="s")
    run = functools.partial(
        pl.kernel,
        mesh=mesh,
        out_type=jax.ShapeDtypeStruct((NUM_HEADS, QT, KT, 8, 128), jnp.float32),
        scratch_types=[
            pltpu.VMEM((2, 1, 8, TBL), jnp.float32),
            pltpu.SemaphoreType.DMA,
        ],
        compiler_params=pltpu.CompilerParams(use_tc_tiling_on_sc=False),
    )(_expand_body)
    return run(tbl)


def kernel(weight, seq_length, key_length):
    delta = key_length - seq_length
    tbl = _build_table(weight, delta)
    out5 = _expand(tbl)
    return out5.transpose(0, 1, 3, 2, 4).reshape(NUM_HEADS, SEQ, KEY)


# speedup vs baseline: 1.0305x; 1.0021x over previous
import functools
import math

import jax
import jax.numpy as jnp
from jax import lax
from jax.experimental import pallas as pl
from jax.experimental.pallas import tpu as pltpu
from jax.experimental.pallas import tpu_sc as plsc

NUM_HEADS = 16
NUM_BUCKETS = 32
MAX_DISTANCE = 128

SEQ = 2048
KEY = 2048
TBL = 4224
TBLP = 4352

QT = SEQ // 8
KT = KEY // 128

NC = 2
NS = 16
NW = NC * NS
QT_PER_W = NUM_HEADS * QT // NW
PAIR_STEPS = QT_PER_W // 2


def _table_body(delta_ref, wt_ref, out_ref):
    delta = delta_ref[0]
    col = lax.broadcasted_iota(jnp.int32, (8, TBLP), 1)
    row = lax.broadcasted_iota(jnp.int32, (8, TBLP), 0)
    d = col + (row - 8) - (KEY - 1) + delta
    half = NUM_BUCKETS // 2
    rel = jnp.where(d > 0, half, 0)
    ad = jnp.abs(d)
    max_exact = half // 2
    rp_safe = jnp.maximum(ad, 1)
    lg = max_exact + (
        jnp.log(rp_safe.astype(jnp.float32) / max_exact)
        / math.log(MAX_DISTANCE / max_exact)
        * (half - max_exact)
    ).astype(jnp.int32)
    lg = jnp.minimum(lg, half - 1)
    bkt = rel + jnp.where(ad < max_exact, ad, lg)
    bi = lax.broadcasted_iota(jnp.int32, (NUM_BUCKETS, TBLP), 0)
    for s in range(8):
        onehot = (bkt[s : s + 1, :] == bi).astype(jnp.float32)
        vals = lax.dot_general(
            wt_ref[...], onehot,
            (((1,), (0,)), ((), ())),
            preferred_element_type=jnp.float32,
            precision=lax.Precision.HIGHEST,
        )
        out_ref[s] = vals


def _build_table(weight, delta):
    wt = weight.T
    delta_arr = jnp.asarray(delta, jnp.int32).reshape(1)
    return pl.pallas_call(
        _table_body,
        out_shape=jax.ShapeDtypeStruct((8, NUM_HEADS, TBLP), jnp.float32),
        in_specs=[
            pl.BlockSpec(memory_space=pltpu.SMEM),
            pl.BlockSpec(memory_space=pltpu.VMEM),
        ],
        out_specs=pl.BlockSpec(memory_space=pltpu.VMEM),
    )(delta_arr, wt)


def _expand_body(tbl_hbm, out_hbm, tbl_v, sem):
    wid = lax.axis_index("s") * NC + lax.axis_index("c")
    h = wid // 2
    qt0 = (wid % 2) * QT_PER_W
    stage = []
    for g in range(2):
        for u in range(8):
            stage.append(
                pltpu.async_copy(
                    tbl_hbm.at[7 - u, h, pl.ds(8 - 8 * g, TBL)],
                    tbl_v.at[g, 0, u],
                    sem,
                )
            )
    for c in stage:
        c.wait()

    def step(t, carry):
        copies = []
        for p in range(2):
            qtg = qt0 + 4 * t + 2 * p
            base = (KEY - 8) - 8 * qtg
            for kt in range(KT):
                copies.append(
                    pltpu.async_copy(
                        tbl_v.at[:, :, :, pl.ds(base + 128 * kt, 128)],
                        out_hbm.at[h, pl.ds(qtg, 2), pl.ds(kt, 1)],
                        sem,
                    )
                )
        for c in copies:
            c.wait()
        return carry

    lax.fori_loop(0, PAIR_STEPS // 2, step, 0)


def _expand(tbl):
    mesh = plsc.VectorSubcoreMesh(core_axis_name="c", subcore_axis_name="s")
    run = functools.partial(
        pl.kernel,
        mesh=mesh,
        out_type=jax.ShapeDtypeStruct((NUM_HEADS, QT, KT, 8, 128), jnp.float32),
        scratch_types=[
            pltpu.VMEM((2, 1, 8, TBL), jnp.float32),
            pltpu.SemaphoreType.DMA,
        ],
        compiler_params=pltpu.CompilerParams(use_tc_tiling_on_sc=False),
    )(_expand_body)
    return run(tbl)


def kernel(weight, seq_length, key_length):
    delta = key_length - seq_length
    tbl = _build_table(weight, delta)
    out5 = _expand(tbl)
    return out5.transpose(0, 1, 3, 2, 4).reshape(NUM_HEADS, SEQ, KEY)
